# baseline (speedup 1.0000x reference)
import functools

import jax
import jax.numpy as jnp
from jax import lax
from jax.experimental import pallas as pl
from jax.experimental.pallas import tpu as pltpu
from jax.experimental.pallas import tpu_sc as plsc

N = 4096
DIN = 384
DH = 512
TK = 6
BR = 256
NB_BLK = N // BR

_NC = 2
_NS = 16
_NW = _NC * _NS
_B = N * TK
_BPW = _B // _NW
_CH = 96
_NCHUNK = _BPW // _CH


def _leaky(x):
    return jnp.where(x >= 0, x, 0.01 * x)


def _dot(a, b):
    return jnp.dot(a, b, preferred_element_type=jnp.float32)


def _ka_body(data_ref, fc1w_ref, fc1b_ref, whw_ref, whb_ref, wtw_ref, wtb_ref,
             eh_ref, et_ref, prob_ref, idx_ref,
             h1_s, eh_s, et_s, csum_s):
    i = pl.program_id(0)

    @pl.when(i < NB_BLK)
    def _phase0():
        h = _leaky(_dot(data_ref[...], fc1w_ref[...]) + fc1b_ref[...])
        h1_s[pl.ds(i * BR, BR), :] = h

        @pl.when(i == 0)
        def _():
            csum_s[...] = jnp.zeros_like(csum_s)

        csum_s[...] += jnp.sum(h, axis=0, keepdims=True)

    @pl.when(jnp.logical_and(i >= NB_BLK, i < 2 * NB_BLK))
    def _phase1():
        j = i - NB_BLK
        x = (h1_s[pl.ds(j * BR, BR), :] + csum_s[...] * (1.0 / N)) * 0.5
        eh_s[pl.ds(j * BR, BR), :] = _dot(x, whw_ref[...]) + whb_ref[...]
        et_s[pl.ds(j * BR, BR), :] = _dot(x, wtw_ref[...]) + wtb_ref[...]

    @pl.when(i >= 2 * NB_BLK)
    def _phase2():
        j = i - 2 * NB_BLK
        eh = eh_s[pl.ds(j * BR, BR), :]
        et = et_s[pl.ds(j * BR, BR), :]
        scale = DH ** (-0.5)
        logits = lax.dot_general(eh * scale, et_s[...],
                                 (((1,), (1,)), ((), ())),
                                 precision=lax.Precision.HIGHEST,
                                 preferred_element_type=jnp.float32)
        iota = lax.broadcasted_iota(jnp.int32, logits.shape, 1)
        vals, idxs = [], []
        for _ in range(TK):
            m = jnp.max(logits, axis=1, keepdims=True)
            jj = jnp.min(jnp.where(logits >= m, iota, N), axis=1, keepdims=True)
            vals.append(m)
            idxs.append(jj)
            logits = jnp.where(iota == jj, -jnp.inf, logits)
        v = jnp.concatenate(vals, axis=1)
        ji = jnp.concatenate(idxs, axis=1)
        e = jnp.exp(v - v[:, 0:1])
        prob_ref[...] = e / jnp.sum(e, axis=1, keepdims=True)
        idx_ref[...] = ji
        eh_ref[...] = eh
        et_ref[...] = et


def _sc_gather_body(table_hbm, idx_hbm, out_hbm, idx_v, b0, b1,
                    sg0, sg1, ss0, ss1):
    wid = lax.axis_index("s") * _NC + lax.axis_index("c")
    base = wid * _BPW
    pltpu.sync_copy(idx_hbm.at[pl.ds(base, _BPW)], idx_v)
    bufs, gsem, ssem = (b0, b1), (sg0, sg1), (ss0, ss1)
    gh = [None, None]
    sh = [None, None]
    gh[0] = pltpu.async_copy(table_hbm.at[idx_v.at[pl.ds(0, _CH)]], b0, sg0)
    for c in range(_NCHUNK):
        cur = c & 1
        nxt = 1 - cur
        if c + 1 < _NCHUNK:
            if sh[nxt] is not None:
                sh[nxt].wait()
            gh[nxt] = pltpu.async_copy(
                table_hbm.at[idx_v.at[pl.ds((c + 1) * _CH, _CH)]],
                bufs[nxt], gsem[nxt])
        gh[cur].wait()
        sh[cur] = pltpu.async_copy(
            bufs[cur], out_hbm.at[pl.ds(base + c * _CH, _CH)], ssem[cur])
    sh[0].wait()
    sh[1].wait()


def _gather_rows(table, idx):
    mesh = plsc.VectorSubcoreMesh(
        core_axis_name="c", subcore_axis_name="s",
        num_cores=_NC, num_subcores=_NS)
    f = functools.partial(
        pl.kernel, mesh=mesh,
        out_type=jax.ShapeDtypeStruct((_B, DH), jnp.float32),
        scratch_types=[
            pltpu.VMEM((_BPW,), jnp.int32),
            pltpu.VMEM((_CH, DH), jnp.float32),
            pltpu.VMEM((_CH, DH), jnp.float32),
            pltpu.SemaphoreType.DMA,
            pltpu.SemaphoreType.DMA,
            pltpu.SemaphoreType.DMA,
            pltpu.SemaphoreType.DMA,
        ],
    )(_sc_gather_body)
    return f(table, idx)


def _kc_body(nb_ref, eh_ref, p_ref, l1w_ref, l1b_ref, l2w_ref, l2b_ref,
             a1w_ref, a1b_ref, a2w_ref, a2b_ref,
             ng_ref, nbeta_ref, fcw_ref, fcb_ref,
             lg_ref, yp_ref, yh_ref, emb_s, g_s):
    i = pl.program_id(0)

    @pl.when(i < NB_BLK)
    def _combine():
        Nb = nb_ref[...]
        eh = eh_ref[...]
        p3 = p_ref[...][:, :, None]
        eh3 = eh[:, None, :]
        eh_r = p3 * Nb + (1.0 - p3) * eh3
        gate = jnp.tanh(eh3 + eh_r)
        ka = jnp.sum(Nb, axis=2) * jnp.sum(gate, axis=2)
        m = jnp.max(ka, axis=1, keepdims=True)
        e = jnp.exp(ka - m)
        kp = e / jnp.sum(e, axis=1, keepdims=True)
        eNh = jnp.sum(kp[:, :, None] * Nb, axis=1)
        s = _leaky(_dot(eh + eNh, l1w_ref[...]) + l1b_ref[...])
        bi = _leaky(_dot(eh * eNh, l2w_ref[...]) + l2b_ref[...])
        emb = s + bi
        emb_s[pl.ds(i * BR, BR), :] = emb
        a1 = _leaky(_dot(emb, a1w_ref[...]) + a1b_ref[...])
        g_s[pl.ds(i * BR, BR), :] = _dot(a1, a2w_ref[...]) + a2b_ref[...]

    @pl.when(i == NB_BLK)
    def _readout():
        h = emb_s[...]
        g = g_s[...]
        m = jnp.max(g, axis=0, keepdims=True)
        e = jnp.exp(g - m)
        a = e / jnp.sum(e, axis=0, keepdims=True)
        hr = jnp.sum(a * h, axis=0, keepdims=True)
        mu = jnp.mean(hr, axis=1, keepdims=True)
        var = jnp.mean((hr - mu) ** 2, axis=1, keepdims=True)
        hn = (hr - mu) / jnp.sqrt(var + 1e-5) * ng_ref[...] + nbeta_ref[...]
        lg = _dot(hn, fcw_ref[...]) + fcb_ref[...]
        lg_ref[...] = lg
        mm = jnp.max(lg, axis=1, keepdims=True)
        ee = jnp.exp(lg - mm)
        yp_ref[...] = ee / jnp.sum(ee, axis=1, keepdims=True)
        yh_ref[...] = jnp.where(lg[:, 1:2] > lg[:, 0:1], 1, 0).astype(jnp.int32)


def kernel(data, CT_data, fc1_W, fc1_b, Wh_W, Wh_b, Wt_W, Wt_b,
           lin1_W, lin1_b, lin2_W, lin2_b, att1_W, att1_b, att2_W, att2_b,
           norm_g, norm_beta, fc_W, fc_b):
    del CT_data
    x0 = jnp.squeeze(data, axis=0)
    r2 = lambda v: v.reshape(1, -1)
    full = lambda a, b: pl.BlockSpec((a, b), lambda i: (0, 0))
    p2rows = lambda b: pl.BlockSpec(
        (BR, b), lambda i: (jnp.where(i < 2 * NB_BLK, 0, i - 2 * NB_BLK), 0))

    e_h, e_t, probs, idx = pl.pallas_call(
        _ka_body,
        grid=(3 * NB_BLK,),
        in_specs=[pl.BlockSpec((BR, DIN), lambda i: (jnp.minimum(i, NB_BLK - 1), 0)),
                  full(DIN, DH), full(1, DH),
                  full(DH, DH), full(1, DH),
                  full(DH, DH), full(1, DH)],
        out_specs=[p2rows(DH), p2rows(DH), p2rows(TK), p2rows(TK)],
        out_shape=[jax.ShapeDtypeStruct((N, DH), jnp.float32),
                   jax.ShapeDtypeStruct((N, DH), jnp.float32),
                   jax.ShapeDtypeStruct((N, TK), jnp.float32),
                   jax.ShapeDtypeStruct((N, TK), jnp.int32)],
        scratch_shapes=[pltpu.VMEM((N, DH), jnp.float32),
                        pltpu.VMEM((N, DH), jnp.float32),
                        pltpu.VMEM((N, DH), jnp.float32),
                        pltpu.VMEM((1, DH), jnp.float32)],
    )(x0, fc1_W, r2(fc1_b), Wh_W, r2(Wh_b), Wt_W, r2(Wt_b))

    nb = _gather_rows(e_t, idx.reshape(_B))
    nb3 = nb.reshape(N, TK, DH)

    rows16 = lambda b: pl.BlockSpec((BR, b), lambda i: (jnp.minimum(i, NB_BLK - 1), 0))
    out01 = lambda a, b: pl.BlockSpec((a, b), lambda i: (0, 0))

    logits, y_prob, y_hat = pl.pallas_call(
        _kc_body,
        grid=(NB_BLK + 1,),
        in_specs=[pl.BlockSpec((BR, TK, DH),
                               lambda i: (jnp.minimum(i, NB_BLK - 1), 0, 0)),
                  rows16(DH), rows16(TK),
                  full(DH, DH), full(1, DH), full(DH, DH), full(1, DH),
                  full(DH, DH // 2), full(1, DH // 2), full(DH // 2, 1),
                  full(1, 1),
                  full(1, DH), full(1, DH), full(DH, 2), full(1, 2)],
        out_specs=[out01(1, 2), out01(1, 2), out01(1, 1)],
        out_shape=[jax.ShapeDtypeStruct((1, 2), jnp.float32),
                   jax.ShapeDtypeStruct((1, 2), jnp.float32),
                   jax.ShapeDtypeStruct((1, 1), jnp.int32)],
        scratch_shapes=[pltpu.VMEM((N, DH), jnp.float32),
                        pltpu.VMEM((N, 1), jnp.float32)],
    )(nb3, e_h, probs, lin1_W, r2(lin1_b), lin2_W, r2(lin2_b),
      att1_W, r2(att1_b), att2_W, r2(att2_b),
      r2(norm_g), r2(norm_beta), fc_W, r2(fc_b))

    return (logits, y_prob, y_hat)


# speedup vs baseline: 1.4161x; 1.4161x over previous
import functools

import jax
import jax.numpy as jnp
from jax import lax
from jax.experimental import pallas as pl
from jax.experimental.pallas import tpu as pltpu
from jax.experimental.pallas import tpu_sc as plsc

N = 4096
DIN = 384
DH = 512
TK = 6
BR = 256
NBB = N // BR
NH = N // 2
NBH = NH // BR

_NC = 2
_NS = 16
_NW = _NC * _NS
_BH = NH * TK
_BPW = _BH // _NW
_CH = 96
_NCHUNK = _BPW // _CH


def _leaky(x):
    return jnp.where(x >= 0, x, 0.01 * x)


def _dot(a, b):
    return jnp.dot(a, b, preferred_element_type=jnp.float32)


def _topk_block(eh, et_full):
    scale = DH ** (-0.5)
    logits = lax.dot_general(eh * scale, et_full,
                             (((1,), (1,)), ((), ())),
                             preferred_element_type=jnp.float32)
    iota = lax.broadcasted_iota(jnp.int32, logits.shape, 1)
    vals, idxs = [], []
    for _ in range(TK):
        m = jnp.max(logits, axis=1, keepdims=True)
        jj = jnp.min(jnp.where(logits >= m, iota, N), axis=1, keepdims=True)
        vals.append(m)
        idxs.append(jj)
        logits = jnp.where(iota == jj, -jnp.inf, logits)
    v = jnp.concatenate(vals, axis=1)
    ji = jnp.concatenate(idxs, axis=1)
    e = jnp.exp(v - v[:, 0:1])
    return e / jnp.sum(e, axis=1, keepdims=True), ji


def _ka_body(data_ref, fc1w_ref, fc1b_ref, whw_ref, whb_ref, wtw_ref, wtb_ref,
             eh_ref, et_ref, prob_ref, idx_ref,
             h1_s, eh_s, et_s, csum_s):
    i = pl.program_id(0)

    @pl.when(i < NBB)
    def _phase0():
        h = _leaky(_dot(data_ref[...], fc1w_ref[...]) + fc1b_ref[...])
        h1_s[pl.ds(i * BR, BR), :] = h

        @pl.when(i == 0)
        def _():
            csum_s[...] = jnp.zeros_like(csum_s)

        csum_s[...] += jnp.sum(h, axis=0, keepdims=True)

    @pl.when(jnp.logical_and(i >= NBB, i < 2 * NBB))
    def _phase1():
        j = i - NBB
        x = (h1_s[pl.ds(j * BR, BR), :] + csum_s[...] * (1.0 / N)) * 0.5
        eh = _dot(x, whw_ref[...]) + whb_ref[...]
        et = _dot(x, wtw_ref[...]) + wtb_ref[...]
        eh_s[pl.ds(j * BR, BR), :] = eh
        et_s[pl.ds(j * BR, BR), :] = et
        eh_ref[...] = eh
        et_ref[...] = et

    @pl.when(i >= 2 * NBB)
    def _phase2():
        j = i - 2 * NBB
        p, ji = _topk_block(eh_s[pl.ds(j * BR, BR), :], et_s[...])
        prob_ref[...] = p
        idx_ref[...] = ji


def _kb_body(eh_ref, et_ref, prob_ref, idx_ref):
    p, ji = _topk_block(eh_ref[...], et_ref[...])
    prob_ref[...] = p
    idx_ref[...] = ji


def _sc_gather_body(table_hbm, idx_hbm, out_hbm, idx_v, b0, b1,
                    sg0, sg1, ss0, ss1):
    wid = lax.axis_index("s") * _NC + lax.axis_index("c")
    base = wid * _BPW
    pltpu.sync_copy(idx_hbm.at[pl.ds(base, _BPW)], idx_v)
    bufs, gsem, ssem = (b0, b1), (sg0, sg1), (ss0, ss1)
    gh = [None, None]
    sh = [None, None]
    gh[0] = pltpu.async_copy(table_hbm.at[idx_v.at[pl.ds(0, _CH)]], b0, sg0)
    for c in range(_NCHUNK):
        cur = c & 1
        nxt = 1 - cur
        if c + 1 < _NCHUNK:
            if sh[nxt] is not None:
                sh[nxt].wait()
            gh[nxt] = pltpu.async_copy(
                table_hbm.at[idx_v.at[pl.ds((c + 1) * _CH, _CH)]],
                bufs[nxt], gsem[nxt])
        gh[cur].wait()
        sh[cur] = pltpu.async_copy(
            bufs[cur], out_hbm.at[pl.ds(base + c * _CH, _CH)], ssem[cur])
    sh[0].wait()
    sh[1].wait()


def _gather_rows(table, idx):
    mesh = plsc.VectorSubcoreMesh(
        core_axis_name="c", subcore_axis_name="s",
        num_cores=_NC, num_subcores=_NS)
    f = functools.partial(
        pl.kernel, mesh=mesh,
        out_type=jax.ShapeDtypeStruct((_BH, DH), jnp.float32),
        scratch_types=[
            pltpu.VMEM((_BPW,), jnp.int32),
            pltpu.VMEM((_CH, DH), jnp.float32),
            pltpu.VMEM((_CH, DH), jnp.float32),
            pltpu.SemaphoreType.DMA,
            pltpu.SemaphoreType.DMA,
            pltpu.SemaphoreType.DMA,
            pltpu.SemaphoreType.DMA,
        ],
    )(_sc_gather_body)
    return f(table, idx)


def _combine_block(Nb, eh, p, l1w, l1b, l2w, l2b, a1w, a1b, a2w, a2b):
    p3 = p[:, :, None]
    eh3 = eh[:, None, :]
    eh_r = p3 * Nb + (1.0 - p3) * eh3
    gate = jnp.tanh(eh3 + eh_r)
    ka = jnp.sum(Nb, axis=2) * jnp.sum(gate, axis=2)
    m = jnp.max(ka, axis=1, keepdims=True)
    e = jnp.exp(ka - m)
    kp = e / jnp.sum(e, axis=1, keepdims=True)
    eNh = jnp.sum(kp[:, :, None] * Nb, axis=1)
    s = _leaky(_dot(eh + eNh, l1w) + l1b)
    bi = _leaky(_dot(eh * eNh, l2w) + l2b)
    emb = s + bi
    a1 = _leaky(_dot(emb, a1w) + a1b)
    g = _dot(a1, a2w) + a2b
    return emb, g


def _kca_body(nb_ref, eh_ref, p_ref, l1w_ref, l1b_ref, l2w_ref, l2b_ref,
              a1w_ref, a1b_ref, a2w_ref, a2b_ref, emb_ref, g_ref):
    emb, g = _combine_block(
        nb_ref[...], eh_ref[...], p_ref[...],
        l1w_ref[...], l1b_ref[...], l2w_ref[...], l2b_ref[...],
        a1w_ref[...], a1b_ref[...], a2w_ref[...], a2b_ref[...])
    emb_ref[...] = emb
    g_ref[...] = g


def _kcb_body(nb_ref, eh_ref, p_ref, l1w_ref, l1b_ref, l2w_ref, l2b_ref,
              a1w_ref, a1b_ref, a2w_ref, a2b_ref,
              emb0_ref, g0_ref, ng_ref, nbeta_ref, fcw_ref, fcb_ref,
              lg_ref, yp_ref, yh_ref, emb_s, g_s):
    i = pl.program_id(0)

    @pl.when(i < NBH)
    def _combine():
        emb, g = _combine_block(
            nb_ref[...], eh_ref[...], p_ref[...],
            l1w_ref[...], l1b_ref[...], l2w_ref[...], l2b_ref[...],
            a1w_ref[...], a1b_ref[...], a2w_ref[...], a2b_ref[...])
        emb_s[pl.ds(i * BR, BR), :] = emb
        g_s[pl.ds(i * BR, BR), :] = g

    @pl.when(i == NBH)
    def _readout():
        h0 = emb0_ref[...]
        g0 = g0_ref[...]
        h1 = emb_s[...]
        g1 = g_s[...]
        m = jnp.maximum(jnp.max(g0), jnp.max(g1))
        e0 = jnp.exp(g0 - m)
        e1 = jnp.exp(g1 - m)
        denom = jnp.sum(e0) + jnp.sum(e1)
        hr = (jnp.sum(e0 * h0, axis=0, keepdims=True)
              + jnp.sum(e1 * h1, axis=0, keepdims=True)) / denom
        mu = jnp.mean(hr, axis=1, keepdims=True)
        var = jnp.mean((hr - mu) ** 2, axis=1, keepdims=True)
        hn = (hr - mu) / jnp.sqrt(var + 1e-5) * ng_ref[...] + nbeta_ref[...]
        lg = _dot(hn, fcw_ref[...]) + fcb_ref[...]
        lg_ref[...] = lg
        mm = jnp.max(lg, axis=1, keepdims=True)
        ee = jnp.exp(lg - mm)
        yp_ref[...] = ee / jnp.sum(ee, axis=1, keepdims=True)
        yh_ref[...] = jnp.where(lg[:, 1:2] > lg[:, 0:1], 1, 0).astype(jnp.int32)


def kernel(data, CT_data, fc1_W, fc1_b, Wh_W, Wh_b, Wt_W, Wt_b,
           lin1_W, lin1_b, lin2_W, lin2_b, att1_W, att1_b, att2_W, att2_b,
           norm_g, norm_beta, fc_W, fc_b):
    del CT_data
    x0 = jnp.squeeze(data, axis=0)
    r2 = lambda v: v.reshape(1, -1)
    full = lambda a, b: pl.BlockSpec((a, b), lambda i: (0, 0))

    ph1rows = lambda b: pl.BlockSpec(
        (BR, b), lambda i: (jnp.clip(i - NBB, 0, NBB - 1), 0))
    ph2rows = lambda b: pl.BlockSpec(
        (BR, b), lambda i: (jnp.clip(i - 2 * NBB, 0, NBH - 1), 0))

    e_h, e_t, probs0, idx0 = pl.pallas_call(
        _ka_body,
        grid=(2 * NBB + NBH,),
        in_specs=[pl.BlockSpec((BR, DIN), lambda i: (jnp.minimum(i, NBB - 1), 0)),
                  full(DIN, DH), full(1, DH),
                  full(DH, DH), full(1, DH),
                  full(DH, DH), full(1, DH)],
        out_specs=[ph1rows(DH), ph1rows(DH), ph2rows(TK), ph2rows(TK)],
        out_shape=[jax.ShapeDtypeStruct((N, DH), jnp.float32),
                   jax.ShapeDtypeStruct((N, DH), jnp.float32),
                   jax.ShapeDtypeStruct((NH, TK), jnp.float32),
                   jax.ShapeDtypeStruct((NH, TK), jnp.int32)],
        scratch_shapes=[pltpu.VMEM((N, DH), jnp.float32),
                        pltpu.VMEM((N, DH), jnp.float32),
                        pltpu.VMEM((N, DH), jnp.float32),
                        pltpu.VMEM((1, DH), jnp.float32)],
    )(x0, fc1_W, r2(fc1_b), Wh_W, r2(Wh_b), Wt_W, r2(Wt_b))

    rowsH = lambda b, off: pl.BlockSpec(
        (BR, b), lambda i: (off + jnp.minimum(i, NBH - 1), 0))

    probs1, idx1 = pl.pallas_call(
        _kb_body,
        grid=(NBH,),
        in_specs=[pl.BlockSpec((BR, DH), lambda i: (NBH + i, 0)),
                  full(N, DH)],
        out_specs=[pl.BlockSpec((BR, TK), lambda i: (i, 0)),
                   pl.BlockSpec((BR, TK), lambda i: (i, 0))],
        out_shape=[jax.ShapeDtypeStruct((NH, TK), jnp.float32),
                   jax.ShapeDtypeStruct((NH, TK), jnp.int32)],
    )(e_h, e_t)

    nb0 = _gather_rows(e_t, idx0.reshape(_BH)).reshape(NH, TK, DH)
    nb1 = _gather_rows(e_t, idx1.reshape(_BH)).reshape(NH, TK, DH)

    wspecs = [full(DH, DH), full(1, DH), full(DH, DH), full(1, DH),
              full(DH, DH // 2), full(1, DH // 2), full(DH // 2, 1),
              full(1, 1)]
    wargs = (lin1_W, r2(lin1_b), lin2_W, r2(lin2_b),
             att1_W, r2(att1_b), att2_W, r2(att2_b))

    emb0, g0 = pl.pallas_call(
        _kca_body,
        grid=(NBH,),
        in_specs=[pl.BlockSpec((BR, TK, DH), lambda i: (i, 0, 0)),
                  pl.BlockSpec((BR, DH), lambda i: (i, 0)),
                  pl.BlockSpec((BR, TK), lambda i: (i, 0))] + wspecs,
        out_specs=[pl.BlockSpec((BR, DH), lambda i: (i, 0)),
                   pl.BlockSpec((BR, 1), lambda i: (i, 0))],
        out_shape=[jax.ShapeDtypeStruct((NH, DH), jnp.float32),
                   jax.ShapeDtypeStruct((NH, 1), jnp.float32)],
    )(nb0, e_h, probs0, *wargs)

    out01 = lambda a, b: pl.BlockSpec((a, b), lambda i: (0, 0))

    logits, y_prob, y_hat = pl.pallas_call(
        _kcb_body,
        grid=(NBH + 1,),
        in_specs=[pl.BlockSpec((BR, TK, DH),
                               lambda i: (jnp.minimum(i, NBH - 1), 0, 0)),
                  rowsH(DH, NBH),
                  pl.BlockSpec((BR, TK), lambda i: (jnp.minimum(i, NBH - 1), 0))]
                 + wspecs
                 + [full(NH, DH), full(NH, 1),
                    full(1, DH), full(1, DH), full(DH, 2), full(1, 2)],
        out_specs=[out01(1, 2), out01(1, 2), out01(1, 1)],
        out_shape=[jax.ShapeDtypeStruct((1, 2), jnp.float32),
                   jax.ShapeDtypeStruct((1, 2), jnp.float32),
                   jax.ShapeDtypeStruct((1, 1), jnp.int32)],
        scratch_shapes=[pltpu.VMEM((NH, DH), jnp.float32),
                        pltpu.VMEM((NH, 1), jnp.float32)],
    )(nb1, e_h, probs1, *wargs, emb0, g0,
      r2(norm_g), r2(norm_beta), fc_W, r2(fc_b))

    return (logits, y_prob, y_hat)
